# baseline (speedup 1.0000x reference)
import jax
import jax.numpy as jnp
from jax.experimental import pallas as pl
from jax.experimental.pallas import tpu as pltpu

N_ENT = 10000
N_REL = 230
N_TIME = 365
N_FACTS = 160000
D = 20


def _cls_body(h_ref, m_ref, w_ref, b_ref, o_ref):
    h = h_ref[0]
    w = w_ref[...]
    m = m_ref[0]
    o_ref[0] = (jnp.sum(h * w, axis=0, keepdims=True) + b_ref[0, 0]) * m


def _classifier(hidden, mem, W_cls, b_cls):
    B = hidden.shape[0]
    hT = hidden.transpose(0, 2, 1)
    memf = mem.astype(jnp.float32)[:, None, :]
    out = pl.pallas_call(
        _cls_body,
        grid=(B,),
        in_specs=[
            pl.BlockSpec((1, D, N_ENT), lambda b: (b, 0, 0)),
            pl.BlockSpec((1, 1, N_ENT), lambda b: (b, 0, 0)),
            pl.BlockSpec((D, 1), lambda b: (0, 0)),
            pl.BlockSpec(memory_space=pltpu.SMEM),
        ],
        out_specs=pl.BlockSpec((1, 1, N_ENT), lambda b: (b, 0, 0)),
        out_shape=jax.ShapeDtypeStruct((B, 1, N_ENT), jnp.float32),
    )(hT, memf, W_cls, b_cls.reshape(1, 1))
    return out[:, 0, :]


def kernel(head, relation, time, example_idx, dataset, rela_embed, time_embed,
           W_att1, W_att2, W_past, W_now, W_future, W_cls, b_cls):
    Bn = head.shape[0]
    valid = jnp.ones((N_FACTS,), dtype=bool).at[example_idx].set(False)
    e_head = dataset[:, 0]
    e_rel = dataset[:, 1]
    e_tail = dataset[:, 2]
    e_time = dataset[:, 3]
    mem = jnp.zeros((Bn, N_ENT), dtype=bool).at[jnp.arange(Bn), head].set(True)
    hidden = jnp.zeros((Bn, N_ENT, D), dtype=jnp.float32)

    def _layer(args):
        h_b, mem_b, qrel_b, t_b = args
        edge_mask = valid & mem_b[e_head]
        h_src = h_b[e_head]
        rel_time = e_time - t_b
        embed_rel = h_src + rela_embed[e_rel] + time_embed[jnp.abs(rel_time)]
        fut = embed_rel @ W_future
        now = embed_rel @ W_now
        past = embed_rel @ W_past
        s = jnp.sign(rel_time)[:, None]
        transformed = jnp.where(s > 0, fut, jnp.where(s == 0, now, past))
        qrel_row = jnp.broadcast_to(rela_embed[qrel_b], (N_FACTS, D))
        att_in = jnp.concatenate([h_src, rela_embed[e_rel], qrel_row], axis=1)
        score = jax.nn.sigmoid(jax.nn.relu(att_in @ W_att1) @ W_att2)
        msg = jnp.where(edge_mask[:, None], score * transformed, 0.0)
        new_h = jax.ops.segment_sum(msg, e_tail, num_segments=N_ENT)
        new_h = jax.nn.leaky_relu(new_h, negative_slope=0.01)
        new_mem = jnp.zeros((N_ENT,), dtype=bool).at[e_tail].max(edge_mask)
        return new_h, new_mem

    for _ in range(3):
        hidden, mem = jax.lax.map(_layer, (hidden, mem, relation, time))

    return _classifier(hidden, mem, W_cls, b_cls)


# speedup vs baseline: 6.9302x; 6.9302x over previous
import functools

import jax
import jax.numpy as jnp
from jax import lax
from jax.experimental import pallas as pl
from jax.experimental.pallas import tpu as pltpu
from jax.experimental.pallas import tpu_sc as plsc

N_ENT = 10000
N_REL = 230
N_TIME = 365
N_FACTS = 160000
B = 64
D = 20

NEP = 10240
C = 128
NFP = 163840
NTILE = 16
CPT = NFP // C // NTILE
BPC = B // 2
RPT = NEP // NTILE
NTP = 368
NRP = 232
GRP = C // 16


def _build_body(acc_ref, w_ref, o_ref):
    a = acc_ref[0]
    h32 = jnp.where(a > 0, a, 0.01 * a)
    o = jnp.dot(h32, w_ref[...], preferred_element_type=jnp.float32)
    memf = (a[:, 20:21] > 0).astype(jnp.float32)
    iot = lax.broadcasted_iota(jnp.int32, (NEP, 128), 1)
    o_ref[0] = o + jnp.where(iot == 126, memf, 0.0)


def _build_tables(acc, w_cat):
    tab = pl.pallas_call(
        _build_body,
        grid=(B,),
        in_specs=[
            pl.BlockSpec((1, NEP, 32), lambda b: (b, 0, 0)),
            pl.BlockSpec((32, 128), lambda b: (0, 0)),
        ],
        out_specs=pl.BlockSpec((1, NEP, 128), lambda b: (b, 0, 0)),
        out_shape=jax.ShapeDtypeStruct((B, NEP, 128), jnp.float32),
    )(acc, w_cat)
    t1 = tab[:, :, :96].reshape(B * NEP * 3, 32)
    t2 = tab[:, :, 96:].reshape(B * NEP, 32)
    return t1, t2


def _cls_body(acc_ref, w_ref, b_ref, o_ref):
    a = acc_ref[0]
    h32 = jnp.where(a > 0, a, 0.01 * a)
    r = jnp.dot(h32, w_ref[...], preferred_element_type=jnp.float32)
    memf = (a[:, 20:21] > 0).astype(jnp.float32)
    o_ref[0] = (r + b_ref[0, 0]) * memf


def _classifier(acc, w_cls_p, b_cls):
    out = pl.pallas_call(
        _cls_body,
        grid=(B,),
        in_specs=[
            pl.BlockSpec((1, NEP, 32), lambda b: (b, 0, 0)),
            pl.BlockSpec((32, 1), lambda b: (0, 0)),
            pl.BlockSpec(memory_space=pltpu.SMEM),
        ],
        out_specs=pl.BlockSpec((1, NEP, 1), lambda b: (b, 0, 0)),
        out_shape=jax.ShapeDtypeStruct((B, NEP, 1), jnp.float32),
    )(acc, w_cls_p, b_cls.reshape(1, 1))
    return out[:, :N_ENT, 0]


def _sc_body(eh_hbm, er_hbm, etl_hbm, etm_hbm, ev_hbm,
             t1_hbm, t2_hbm, rw_hbm, tb_hbm, ar_hbm, st_hbm, w2_hbm,
             out_hbm,
             acc_sh, rw_v, tb_v, ar_v, st_v, w2_v,
             eh_v, er_v, etl_v, etm_v, ev_v, i1_v, i2_v,
             t1r_v, t2r_v, msg_v, zer_v, sem1, sem2):
    cid = lax.axis_index("c")
    sid = lax.axis_index("s")
    zeros16 = jnp.zeros((16,), jnp.float32)

    pltpu.sync_copy(rw_hbm, rw_v)
    pltpu.sync_copy(w2_hbm, w2_v)

    def _zrow(i, c):
        zer_v[i, pl.ds(0, 16)] = zeros16
        zer_v[i, pl.ds(16, 16)] = zeros16
        return c
    lax.fori_loop(0, RPT, _zrow, 0)

    def _mrow(i, c):
        msg_v[i, pl.ds(16, 16)] = zeros16
        return c
    lax.fori_loop(0, C, _mrow, 0)

    my_rows = pl.ds(sid * RPT, RPT)
    pltpu.sync_copy(zer_v, acc_sh.at[my_rows])
    plsc.subcore_barrier()

    iota16 = lax.iota(jnp.int32, 16)

    def _batch(bl, carry):
        b = cid * BPC + bl
        pltpu.sync_copy(tb_hbm.at[b], tb_v)
        pltpu.sync_copy(ar_hbm.at[b], ar_v)
        pltpu.sync_copy(st_hbm.at[b], st_v)
        base1 = b * (NEP * 3)
        base2 = b * NEP

        def _chunk(ci, c2):
            eo = (ci * NTILE + sid) * C
            pltpu.sync_copy(eh_hbm.at[pl.ds(eo, C)], eh_v)
            pltpu.sync_copy(er_hbm.at[pl.ds(eo, C)], er_v)
            pltpu.sync_copy(etl_hbm.at[pl.ds(eo, C)], etl_v)
            pltpu.sync_copy(etm_hbm.at[pl.ds(eo, C)], etm_v)
            pltpu.sync_copy(ev_hbm.at[pl.ds(eo, C)], ev_v)

            def _idx(g, c3):
                o = g * 16
                ehd = eh_v[pl.ds(o, 16)]
                s16 = plsc.load_gather(st_v, [etm_v[pl.ds(o, 16)]])
                i1_v[pl.ds(o, 16)] = ehd * 3 + s16 + base1
                i2_v[pl.ds(o, 16)] = ehd + base2
                return c3
            lax.fori_loop(0, GRP, _idx, 0)

            pltpu.async_copy(t1_hbm.at[i1_v], t1r_v, sem1).wait()
            pltpu.async_copy(t2_hbm.at[i2_v], t2r_v, sem2).wait()

            def _grp(g, c3):
                o = g * 16
                row16 = iota16 + o
                er16 = er_v[pl.ds(o, 16)]
                etm16 = etm_v[pl.ds(o, 16)]
                evf = ev_v[pl.ds(o, 16)]
                s16 = plsc.load_gather(st_v, [etm16])
                rr16 = er16 * 3 + s16
                w2a = w2_v[pl.ds(0, 16)]
                w2b = w2_v[pl.ds(16, 16)]
                accs = jnp.zeros((16,), jnp.float32)
                for k in range(30):
                    ck = jnp.full((16,), k, jnp.int32)
                    a = (plsc.load_gather(t2r_v, [row16, ck])
                         + plsc.load_gather(ar_v, [er16, ck]))
                    wk = w2a[k] if k < 16 else w2b[k - 16]
                    accs = accs + jnp.maximum(a, 0.0) * wk
                sig = 1.0 / (1.0 + jnp.exp(-accs))
                memf = plsc.load_gather(t2r_v, [row16, jnp.full((16,), 30, jnp.int32)])
                mk = memf * evf
                sm = sig * mk
                for k in range(20):
                    ck = jnp.full((16,), k, jnp.int32)
                    tr = (plsc.load_gather(t1r_v, [row16, ck])
                          + plsc.load_gather(rw_v, [rr16, ck])
                          + plsc.load_gather(tb_v, [etm16, ck]))
                    plsc.store_scatter(msg_v, [row16, ck], tr * sm)
                plsc.store_scatter(msg_v, [row16, jnp.full((16,), 20, jnp.int32)], mk)
                return c3
            lax.fori_loop(0, GRP, _grp, 0)

            pltpu.sync_copy(msg_v, acc_sh.at[etl_v], add=True)
            return c2
        lax.fori_loop(0, CPT, _chunk, 0)

        plsc.subcore_barrier()
        pltpu.sync_copy(acc_sh.at[my_rows], out_hbm.at[b, my_rows])
        pltpu.sync_copy(zer_v, acc_sh.at[my_rows])
        plsc.subcore_barrier()
        return carry
    lax.fori_loop(0, BPC, _batch, 0)


_sc_edges = functools.partial(
    pl.kernel,
    _sc_body,
    out_type=jax.ShapeDtypeStruct((B, NEP, 32), jnp.float32),
    mesh=plsc.VectorSubcoreMesh(core_axis_name="c", subcore_axis_name="s"),
    compiler_params=pltpu.CompilerParams(
        needs_layout_passes=False, use_tc_tiling_on_sc=False),
    scratch_types=[
        pltpu.VMEM_SHARED((NEP, 32), jnp.float32),
        pltpu.VMEM((3 * NRP, 20), jnp.float32),
        pltpu.VMEM((NTP, 20), jnp.float32),
        pltpu.VMEM((NRP, 30), jnp.float32),
        pltpu.VMEM((NTP,), jnp.int32),
        pltpu.VMEM((32,), jnp.float32),
        pltpu.VMEM((C,), jnp.int32),
        pltpu.VMEM((C,), jnp.int32),
        pltpu.VMEM((C,), jnp.int32),
        pltpu.VMEM((C,), jnp.int32),
        pltpu.VMEM((C,), jnp.float32),
        pltpu.VMEM((C,), jnp.int32),
        pltpu.VMEM((C,), jnp.int32),
        pltpu.VMEM((C, 32), jnp.float32),
        pltpu.VMEM((C, 32), jnp.float32),
        pltpu.VMEM((C, 32), jnp.float32),
        pltpu.VMEM((RPT, 32), jnp.float32),
        pltpu.SemaphoreType.DMA,
        pltpu.SemaphoreType.DMA,
    ],
)()


def kernel(head, relation, time, example_idx, dataset, rela_embed, time_embed,
           W_att1, W_att2, W_past, W_now, W_future, W_cls, b_cls):
    f32 = jnp.float32
    i32 = jnp.int32

    e_head = dataset[:, 0].astype(i32)
    e_rel = dataset[:, 1].astype(i32)
    e_tail = dataset[:, 2].astype(i32)
    e_time = dataset[:, 3].astype(i32)
    validf = jnp.ones((N_FACTS,), f32).at[example_idx].set(0.0)

    pad = NFP - N_FACTS
    ehp = jnp.pad(e_head, (0, pad))
    erp = jnp.pad(e_rel, (0, pad))
    etlp = jnp.pad(e_tail, (0, pad))
    etmp = jnp.pad(e_time, (0, pad))
    evp = jnp.pad(validf, (0, pad))

    Ws = jnp.stack([W_past, W_now, W_future])
    W1h, W1r, W1q = W_att1[:D], W_att1[D:2 * D], W_att1[2 * D:]

    rw = jnp.einsum('rd,sde->rse', rela_embed, Ws).reshape((N_REL + 1) * 3, D)
    rw = jnp.pad(rw, ((0, 3 * NRP - (N_REL + 1) * 3), (0, 0)))

    dtt = jnp.arange(N_TIME)[None, :] - time[:, None]
    sidx = (jnp.sign(dtt) + 1).astype(i32)
    te_abs = time_embed[jnp.abs(dtt)]
    tb = jnp.einsum('btd,btde->bte', te_abs, Ws[sidx])
    tb = jnp.pad(tb, ((0, 0), (0, NTP - N_TIME), (0, 0)))
    st = jnp.pad(sidx, ((0, 0), (0, NTP - N_TIME)))

    ar = rela_embed @ W1r + (rela_embed[relation] @ W1q)[:, None, :]
    ar = jnp.pad(ar, ((0, 0), (0, NRP - (N_REL + 1)), (0, 0)))

    w2 = jnp.pad(W_att2[:, 0], (0, 2))

    w_cat = jnp.zeros((32, 128), f32)
    w_cat = w_cat.at[:D, 0:20].set(W_past)
    w_cat = w_cat.at[:D, 32:52].set(W_now)
    w_cat = w_cat.at[:D, 64:84].set(W_future)
    w_cat = w_cat.at[:D, 96:126].set(W1h)

    w_cls_p = jnp.zeros((32, 1), f32).at[:D].set(W_cls)

    acc = jnp.zeros((B, NEP, 32), f32).at[jnp.arange(B), head, 20].set(1.0)

    for _ in range(3):
        t1, t2 = _build_tables(acc, w_cat)
        acc = _sc_edges(ehp, erp, etlp, etmp, evp, t1, t2, rw, tb, ar, st, w2)

    return _classifier(acc, w_cls_p, b_cls)


# speedup vs baseline: 9.5795x; 1.3823x over previous
import functools

import jax
import jax.numpy as jnp
from jax import lax
from jax.experimental import pallas as pl
from jax.experimental.pallas import tpu as pltpu
from jax.experimental.pallas import tpu_sc as plsc

N_ENT = 10000
N_REL = 230
N_TIME = 365
N_FACTS = 160000
B = 64
D = 20

NEP = 10240
C = 128
NFP = 163840
NTILE = 16
CPT = NFP // C // NTILE
BPC = B // 2
RPT = NEP // NTILE
NTP = 368
NRP = 232
GRP = C // 16


def _build_body(acc_ref, w_ref, o_ref):
    a = acc_ref[0]
    h32 = jnp.where(a > 0, a, 0.01 * a)
    o = jnp.dot(h32, w_ref[...], preferred_element_type=jnp.float32)
    memf = (a[:, 20:21] > 0).astype(jnp.float32)
    iot = lax.broadcasted_iota(jnp.int32, (NEP, 128), 1)
    o_ref[0] = o + jnp.where(iot == 126, memf, 0.0)


def _build_tables(acc, w_cat):
    tab = pl.pallas_call(
        _build_body,
        grid=(B,),
        in_specs=[
            pl.BlockSpec((1, NEP, 32), lambda b: (b, 0, 0)),
            pl.BlockSpec((32, 128), lambda b: (0, 0)),
        ],
        out_specs=pl.BlockSpec((1, NEP, 128), lambda b: (b, 0, 0)),
        out_shape=jax.ShapeDtypeStruct((B, NEP, 128), jnp.float32),
    )(acc, w_cat)
    t1 = tab[:, :, :96].reshape(B * NEP * 3, 32)
    t2 = tab[:, :, 96:].reshape(B * NEP, 32)
    return t1, t2


def _cls_body(acc_ref, w_ref, b_ref, o_ref):
    a = acc_ref[0]
    h32 = jnp.where(a > 0, a, 0.01 * a)
    r = jnp.dot(h32, w_ref[...], preferred_element_type=jnp.float32)
    memf = (a[:, 20:21] > 0).astype(jnp.float32)
    o_ref[0] = (r + b_ref[0, 0]) * memf


def _classifier(acc, w_cls_p, b_cls):
    out = pl.pallas_call(
        _cls_body,
        grid=(B,),
        in_specs=[
            pl.BlockSpec((1, NEP, 32), lambda b: (b, 0, 0)),
            pl.BlockSpec((32, 1), lambda b: (0, 0)),
            pl.BlockSpec(memory_space=pltpu.SMEM),
        ],
        out_specs=pl.BlockSpec((1, NEP, 1), lambda b: (b, 0, 0)),
        out_shape=jax.ShapeDtypeStruct((B, NEP, 1), jnp.float32),
    )(acc, w_cls_p, b_cls.reshape(1, 1))
    return out[:, :N_ENT, 0]


def _sc_body(pk_hbm, t1_hbm, t2_hbm, rw_hbm, tb_hbm, ar_hbm, st_hbm, w2_hbm,
             out_hbm,
             acc_sh, rw_v, tb_v, ar_v, st_v, w2_v,
             pk0, pk1, i10, i20, i11, i21, el0, el1,
             t1r0, t2r0, t1r1, t2r1, msg0, msg1, zer_v,
             sg10, sg20, sg11, sg21, ssc0, ssc1):
    cid = lax.axis_index("c")
    sid = lax.axis_index("s")
    zeros16 = jnp.zeros((16,), jnp.float32)

    pltpu.sync_copy(rw_hbm, rw_v)
    pltpu.sync_copy(w2_hbm, w2_v)

    def _zrow(i, c):
        zer_v[i, pl.ds(0, 16)] = zeros16
        zer_v[i, pl.ds(16, 16)] = zeros16
        return c
    lax.fori_loop(0, RPT, _zrow, 0)

    def _mrow(i, c):
        msg0[i, pl.ds(16, 16)] = zeros16
        msg1[i, pl.ds(16, 16)] = zeros16
        return c
    lax.fori_loop(0, C, _mrow, 0)

    my_rows = pl.ds(sid * RPT, RPT)
    pltpu.sync_copy(zer_v, acc_sh.at[my_rows])
    plsc.subcore_barrier()

    iota16 = lax.iota(jnp.int32, 16)

    def _batch(bl, carry):
        b = cid * BPC + bl
        pltpu.sync_copy(tb_hbm.at[b], tb_v)
        pltpu.sync_copy(ar_hbm.at[b], ar_v)
        pltpu.sync_copy(st_hbm.at[b], st_v)
        base1 = b * (NEP * 3)
        base2 = b * NEP

        def _fetch(ci, pk, i1, i2, el, t1r, t2r, s1, s2):
            pltpu.sync_copy(pk_hbm.at[ci * NTILE + sid], pk)

            def _idx(g, c3):
                o = g * 16
                sl = pl.ds(o, 16)
                ehd = pk[0, sl]
                s16 = plsc.load_gather(st_v, [pk[3, sl]])
                i1[sl] = ehd * 3 + s16 + base1
                i2[sl] = ehd + base2
                el[sl] = pk[2, sl]
                return c3
            lax.fori_loop(0, GRP, _idx, 0)
            pltpu.async_copy(t1_hbm.at[i1], t1r, s1)
            pltpu.async_copy(t2_hbm.at[i2], t2r, s2)

        def _wait_g(i1, i2, t1r, t2r, s1, s2):
            pltpu.make_async_copy(t1_hbm.at[i1], t1r, s1).wait()
            pltpu.make_async_copy(t2_hbm.at[i2], t2r, s2).wait()

        def _compute(pk, t1r, t2r, msg):
            def _grp(g, c3):
                o = g * 16
                sl = pl.ds(o, 16)
                row16 = iota16 + o
                er16 = pk[1, sl]
                etm16 = pk[3, sl]
                evf = pk[4, sl].astype(jnp.float32)
                s16 = plsc.load_gather(st_v, [etm16])
                rr16 = er16 * 3 + s16
                w2a = w2_v[pl.ds(0, 16)]
                w2b = w2_v[pl.ds(16, 16)]
                accs = jnp.zeros((16,), jnp.float32)
                for k in range(30):
                    ck = jnp.full((16,), k, jnp.int32)
                    a = (plsc.load_gather(t2r, [row16, ck])
                         + plsc.load_gather(ar_v, [er16, ck]))
                    wk = w2a[k] if k < 16 else w2b[k - 16]
                    accs = accs + jnp.maximum(a, 0.0) * wk
                sig = 1.0 / (1.0 + jnp.exp(-accs))
                memf = plsc.load_gather(t2r, [row16, jnp.full((16,), 30, jnp.int32)])
                mk = memf * evf
                sm = sig * mk
                for k in range(20):
                    ck = jnp.full((16,), k, jnp.int32)
                    tr = (plsc.load_gather(t1r, [row16, ck])
                          + plsc.load_gather(rw_v, [rr16, ck])
                          + plsc.load_gather(tb_v, [etm16, ck]))
                    plsc.store_scatter(msg, [row16, ck], tr * sm)
                plsc.store_scatter(msg, [row16, jnp.full((16,), 20, jnp.int32)], mk)
                return c3
            lax.fori_loop(0, GRP, _grp, 0)

        def _wait_sc(el, msg, s):
            pltpu.make_async_copy(msg, acc_sh.at[el], s).wait()

        _fetch(0, pk0, i10, i20, el0, t1r0, t2r0, sg10, sg20)

        def _pair(j, c2):
            ci0 = 2 * j
            _fetch(ci0 + 1, pk1, i11, i21, el1, t1r1, t2r1, sg11, sg21)
            _wait_g(i10, i20, t1r0, t2r0, sg10, sg20)

            @pl.when(j > 0)
            def _():
                _wait_sc(el0, msg0, ssc0)
            _compute(pk0, t1r0, t2r0, msg0)
            pltpu.async_copy(msg0, acc_sh.at[el0], ssc0, add=True)

            @pl.when(j < CPT // 2 - 1)
            def _():
                _fetch(ci0 + 2, pk0, i10, i20, el0, t1r0, t2r0, sg10, sg20)
            _wait_g(i11, i21, t1r1, t2r1, sg11, sg21)

            @pl.when(j > 0)
            def _():
                _wait_sc(el1, msg1, ssc1)
            _compute(pk1, t1r1, t2r1, msg1)
            pltpu.async_copy(msg1, acc_sh.at[el1], ssc1, add=True)
            return c2
        lax.fori_loop(0, CPT // 2, _pair, 0)
        _wait_sc(el0, msg0, ssc0)
        _wait_sc(el1, msg1, ssc1)

        plsc.subcore_barrier()
        pltpu.sync_copy(acc_sh.at[my_rows], out_hbm.at[b, my_rows])
        pltpu.sync_copy(zer_v, acc_sh.at[my_rows])
        plsc.subcore_barrier()
        return carry
    lax.fori_loop(0, BPC, _batch, 0)


_sc_edges = functools.partial(
    pl.kernel,
    _sc_body,
    out_type=jax.ShapeDtypeStruct((B, NEP, 32), jnp.float32),
    mesh=plsc.VectorSubcoreMesh(core_axis_name="c", subcore_axis_name="s"),
    compiler_params=pltpu.CompilerParams(
        needs_layout_passes=False, use_tc_tiling_on_sc=False),
    scratch_types=[
        pltpu.VMEM_SHARED((NEP, 32), jnp.float32),
        pltpu.VMEM((3 * NRP, 20), jnp.float32),
        pltpu.VMEM((NTP, 20), jnp.float32),
        pltpu.VMEM((NRP, 30), jnp.float32),
        pltpu.VMEM((NTP,), jnp.int32),
        pltpu.VMEM((32,), jnp.float32),
        pltpu.VMEM((5, C), jnp.int32),
        pltpu.VMEM((5, C), jnp.int32),
        pltpu.VMEM((C,), jnp.int32),
        pltpu.VMEM((C,), jnp.int32),
        pltpu.VMEM((C,), jnp.int32),
        pltpu.VMEM((C,), jnp.int32),
        pltpu.VMEM((C,), jnp.int32),
        pltpu.VMEM((C,), jnp.int32),
        pltpu.VMEM((C, 32), jnp.float32),
        pltpu.VMEM((C, 32), jnp.float32),
        pltpu.VMEM((C, 32), jnp.float32),
        pltpu.VMEM((C, 32), jnp.float32),
        pltpu.VMEM((C, 32), jnp.float32),
        pltpu.VMEM((C, 32), jnp.float32),
        pltpu.VMEM((RPT, 32), jnp.float32),
        pltpu.SemaphoreType.DMA,
        pltpu.SemaphoreType.DMA,
        pltpu.SemaphoreType.DMA,
        pltpu.SemaphoreType.DMA,
        pltpu.SemaphoreType.DMA,
        pltpu.SemaphoreType.DMA,
    ],
)()


def kernel(head, relation, time, example_idx, dataset, rela_embed, time_embed,
           W_att1, W_att2, W_past, W_now, W_future, W_cls, b_cls):
    f32 = jnp.float32
    i32 = jnp.int32

    e_head = dataset[:, 0].astype(i32)
    e_rel = dataset[:, 1].astype(i32)
    e_tail = dataset[:, 2].astype(i32)
    e_time = dataset[:, 3].astype(i32)
    validi = jnp.ones((N_FACTS,), i32).at[example_idx].set(0)

    pad = NFP - N_FACTS
    pk = jnp.stack([
        jnp.pad(e_head, (0, pad)),
        jnp.pad(e_rel, (0, pad)),
        jnp.pad(e_tail, (0, pad)),
        jnp.pad(e_time, (0, pad)),
        jnp.pad(validi, (0, pad)),
    ])
    pk = pk.reshape(5, NFP // C, C).transpose(1, 0, 2)

    Ws = jnp.stack([W_past, W_now, W_future])
    W1h, W1r, W1q = W_att1[:D], W_att1[D:2 * D], W_att1[2 * D:]

    rw = jnp.einsum('rd,sde->rse', rela_embed, Ws).reshape((N_REL + 1) * 3, D)
    rw = jnp.pad(rw, ((0, 3 * NRP - (N_REL + 1) * 3), (0, 0)))

    dtt = jnp.arange(N_TIME)[None, :] - time[:, None]
    sidx = (jnp.sign(dtt) + 1).astype(i32)
    te_abs = time_embed[jnp.abs(dtt)]
    tb = jnp.einsum('btd,btde->bte', te_abs, Ws[sidx])
    tb = jnp.pad(tb, ((0, 0), (0, NTP - N_TIME), (0, 0)))
    st = jnp.pad(sidx, ((0, 0), (0, NTP - N_TIME)))

    ar = rela_embed @ W1r + (rela_embed[relation] @ W1q)[:, None, :]
    ar = jnp.pad(ar, ((0, 0), (0, NRP - (N_REL + 1)), (0, 0)))

    w2 = jnp.pad(W_att2[:, 0], (0, 2))

    w_cat = jnp.zeros((32, 128), f32)
    w_cat = w_cat.at[:D, 0:20].set(W_past)
    w_cat = w_cat.at[:D, 32:52].set(W_now)
    w_cat = w_cat.at[:D, 64:84].set(W_future)
    w_cat = w_cat.at[:D, 96:126].set(W1h)

    w_cls_p = jnp.zeros((32, 1), f32).at[:D].set(W_cls)

    acc = jnp.zeros((B, NEP, 32), f32).at[jnp.arange(B), head, 20].set(1.0)

    for _ in range(3):
        t1, t2 = _build_tables(acc, w_cat)
        acc = _sc_edges(pk, t1, t2, rw, tb, ar, st, w2)

    return _classifier(acc, w_cls_p, b_cls)


# speedup vs baseline: 36.3296x; 3.7924x over previous
import functools

import jax
import jax.numpy as jnp
from jax import lax
from jax.experimental import pallas as pl
from jax.experimental.pallas import tpu as pltpu
from jax.experimental.pallas import tpu_sc as plsc

N_ENT = 10000
N_REL = 230
N_TIME = 365
N_FACTS = 160000
B = 64
D = 20

NEP = 10240
C = 128
NFP = 163840
NTILE = 16
CPT = NFP // C // NTILE
BPC = B // 2
RPT = NEP // NTILE
NTP = 368
NRP = 232
GRP = C // 16
ACC_W = 24


def _build_body(acc_ref, w_ref, o_ref):
    a = acc_ref[0]
    h32 = jnp.where(a > 0, a, 0.01 * a)
    o = jnp.dot(h32, w_ref[...], preferred_element_type=jnp.float32)
    memf = (a[:, 20:21] > 0).astype(jnp.float32)
    iot = lax.broadcasted_iota(jnp.int32, (NEP, 128), 1)
    o_ref[0] = o + jnp.where(iot == 126, memf, 0.0)


def _build_tables(acc, w_cat):
    tab = pl.pallas_call(
        _build_body,
        grid=(B,),
        in_specs=[
            pl.BlockSpec((1, NEP, ACC_W), lambda b: (b, 0, 0)),
            pl.BlockSpec((ACC_W, 128), lambda b: (0, 0)),
        ],
        out_specs=pl.BlockSpec((1, NEP, 128), lambda b: (b, 0, 0)),
        out_shape=jax.ShapeDtypeStruct((B, NEP, 128), jnp.float32),
    )(acc, w_cat)
    t1 = tab[:, :, :96].reshape(B * NEP * 3, 32)
    t2 = tab[:, :, 96:].reshape(B * NEP, 32)
    return t1, t2


def _cls_body(acc_ref, w_ref, b_ref, o_ref):
    a = acc_ref[0]
    h32 = jnp.where(a > 0, a, 0.01 * a)
    r = jnp.dot(h32, w_ref[...], preferred_element_type=jnp.float32)
    memf = (a[:, 20:21] > 0).astype(jnp.float32)
    o_ref[0] = (r + b_ref[0, 0]) * memf


def _classifier(acc, w_cls_p, b_cls):
    out = pl.pallas_call(
        _cls_body,
        grid=(B,),
        in_specs=[
            pl.BlockSpec((1, NEP, ACC_W), lambda b: (b, 0, 0)),
            pl.BlockSpec((ACC_W, 1), lambda b: (0, 0)),
            pl.BlockSpec(memory_space=pltpu.SMEM),
        ],
        out_specs=pl.BlockSpec((1, NEP, 1), lambda b: (b, 0, 0)),
        out_shape=jax.ShapeDtypeStruct((B, NEP, 1), jnp.float32),
    )(acc, w_cls_p, b_cls.reshape(1, 1))
    return out[:, :N_ENT, 0]


def _sc_body(pk_hbm, t1_hbm, t2_hbm, rw_hbm, tb_hbm, ar_hbm, st_hbm, w2_hbm,
             memv_hbm, out_hbm,
             acc_sh, rw_v, tb_v, ar_v, st_v, w2_v, memv_v,
             pk0, pk1, i10, i20, i11, i21, el0, el1, mk0, mk1,
             t1r0, t2r0, t1r1, t2r1, msg0, msg1, zer_v, sflg,
             sg10, sg20, sg11, sg21, ssc0, ssc1):
    cid = lax.axis_index("c")
    sid = lax.axis_index("s")
    zeros16 = jnp.zeros((16,), jnp.float32)

    pltpu.sync_copy(rw_hbm, rw_v)
    pltpu.sync_copy(w2_hbm, w2_v)

    def _zrow(i, c):
        zer_v[i, pl.ds(0, 16)] = zeros16
        zer_v[i, pl.ds(8, 16)] = zeros16
        return c
    lax.fori_loop(0, RPT, _zrow, 0)

    def _mrow(i, c):
        msg0[i, pl.ds(8, 16)] = zeros16
        msg1[i, pl.ds(8, 16)] = zeros16
        return c
    lax.fori_loop(0, C, _mrow, 0)

    sflg[2] = 0
    sflg[3] = 0

    my_rows = pl.ds(sid * RPT, RPT)
    pltpu.sync_copy(zer_v, acc_sh.at[my_rows])
    plsc.subcore_barrier()

    iota16 = lax.iota(jnp.int32, 16)

    def _batch(bl, carry):
        b = cid * BPC + bl
        pltpu.sync_copy(tb_hbm.at[b], tb_v)
        pltpu.sync_copy(ar_hbm.at[b], ar_v)
        pltpu.sync_copy(st_hbm.at[b], st_v)
        pltpu.sync_copy(memv_hbm.at[b], memv_v)
        base1 = b * (NEP * 3)
        base2 = b * NEP

        def _fetch(ci, pk, i1, i2, mk, p, t1r, t2r, s1, s2):
            pltpu.sync_copy(pk_hbm.at[ci * NTILE + sid], pk)

            def _idx(g, ma):
                o = g * 16
                sl = pl.ds(o, 16)
                ehd = pk[0, sl]
                s16 = plsc.load_gather(st_v, [pk[3, sl]])
                i1[sl] = ehd * 3 + s16 + base1
                i2[sl] = ehd + base2
                mf = (plsc.load_gather(memv_v, [ehd]) * pk[4, sl]).astype(jnp.float32)
                mk[sl] = mf
                return jnp.maximum(ma, mf)
            ma = lax.fori_loop(0, GRP, _idx, zeros16)
            any_i = (jnp.max(ma) > 0).astype(jnp.int32)
            sflg[p] = any_i

            @pl.when(any_i > 0)
            def _():
                pltpu.async_copy(t1_hbm.at[i1], t1r, s1)
                pltpu.async_copy(t2_hbm.at[i2], t2r, s2)

        def _wait_g(i1, i2, t1r, t2r, s1, s2):
            pltpu.make_async_copy(t1_hbm.at[i1], t1r, s1).wait()
            pltpu.make_async_copy(t2_hbm.at[i2], t2r, s2).wait()

        def _compute(pk, mk, t1r, t2r, msg):
            def _grp(g, c3):
                o = g * 16
                sl = pl.ds(o, 16)
                mk16 = mk[sl]
                anyg = jnp.max(mk16) > 0

                @pl.when(anyg)
                def _():
                    row16 = iota16 + o
                    er16 = pk[1, sl]
                    etm16 = pk[3, sl]
                    s16 = plsc.load_gather(st_v, [etm16])
                    rr16 = er16 * 3 + s16
                    w2a = w2_v[pl.ds(0, 16)]
                    w2b = w2_v[pl.ds(16, 16)]
                    accs = jnp.zeros((16,), jnp.float32)
                    for k in range(30):
                        ck = jnp.full((16,), k, jnp.int32)
                        a = (plsc.load_gather(t2r, [row16, ck])
                             + plsc.load_gather(ar_v, [er16, ck]))
                        wk = w2a[k] if k < 16 else w2b[k - 16]
                        accs = accs + jnp.maximum(a, 0.0) * wk
                    sig = 1.0 / (1.0 + jnp.exp(-accs))
                    sm = sig * mk16
                    for k in range(20):
                        ck = jnp.full((16,), k, jnp.int32)
                        tr = (plsc.load_gather(t1r, [row16, ck])
                              + plsc.load_gather(rw_v, [rr16, ck])
                              + plsc.load_gather(tb_v, [etm16, ck]))
                        plsc.store_scatter(msg, [row16, ck], tr * sm)
                    plsc.store_scatter(msg, [row16, jnp.full((16,), 20, jnp.int32)], mk16)

                @pl.when(jnp.logical_not(anyg))
                def _():
                    for r in range(16):
                        msg[o + r, pl.ds(0, 16)] = zeros16
                        msg[o + r, pl.ds(8, 16)] = zeros16
                return c3
            lax.fori_loop(0, GRP, _grp, 0)

        def _fill_el(pk, el):
            def _f(g, c3):
                sl = pl.ds(g * 16, 16)
                el[sl] = pk[2, sl]
                return c3
            lax.fori_loop(0, GRP, _f, 0)

        def _wait_sc(el, msg, s):
            pltpu.make_async_copy(msg, acc_sh.at[el], s).wait()

        _fetch(0, pk0, i10, i20, mk0, 0, t1r0, t2r0, sg10, sg20)

        def _pair(j, c2):
            ci0 = 2 * j
            _fetch(ci0 + 1, pk1, i11, i21, mk1, 1, t1r1, t2r1, sg11, sg21)
            ga = sflg[0]

            @pl.when(sflg[2] > 0)
            def _():
                _wait_sc(el0, msg0, ssc0)

            @pl.when(ga > 0)
            def _():
                _wait_g(i10, i20, t1r0, t2r0, sg10, sg20)
                _compute(pk0, mk0, t1r0, t2r0, msg0)
                _fill_el(pk0, el0)
                pltpu.async_copy(msg0, acc_sh.at[el0], ssc0, add=True)
            sflg[2] = ga

            @pl.when(j < CPT // 2 - 1)
            def _():
                _fetch(ci0 + 2, pk0, i10, i20, mk0, 0, t1r0, t2r0, sg10, sg20)
            gb = sflg[1]

            @pl.when(sflg[3] > 0)
            def _():
                _wait_sc(el1, msg1, ssc1)

            @pl.when(gb > 0)
            def _():
                _wait_g(i11, i21, t1r1, t2r1, sg11, sg21)
                _compute(pk1, mk1, t1r1, t2r1, msg1)
                _fill_el(pk1, el1)
                pltpu.async_copy(msg1, acc_sh.at[el1], ssc1, add=True)
            sflg[3] = gb
            return c2
        lax.fori_loop(0, CPT // 2, _pair, 0)

        @pl.when(sflg[2] > 0)
        def _():
            _wait_sc(el0, msg0, ssc0)

        @pl.when(sflg[3] > 0)
        def _():
            _wait_sc(el1, msg1, ssc1)
        sflg[2] = 0
        sflg[3] = 0

        plsc.subcore_barrier()
        pltpu.sync_copy(acc_sh.at[my_rows], out_hbm.at[b, my_rows])
        pltpu.sync_copy(zer_v, acc_sh.at[my_rows])
        plsc.subcore_barrier()
        return carry
    lax.fori_loop(0, BPC, _batch, 0)


_sc_edges = functools.partial(
    pl.kernel,
    _sc_body,
    out_type=jax.ShapeDtypeStruct((B, NEP, ACC_W), jnp.float32),
    mesh=plsc.VectorSubcoreMesh(core_axis_name="c", subcore_axis_name="s"),
    compiler_params=pltpu.CompilerParams(
        needs_layout_passes=False, use_tc_tiling_on_sc=False),
    scratch_types=[
        pltpu.VMEM_SHARED((NEP, ACC_W), jnp.float32),
        pltpu.VMEM((3 * NRP, 20), jnp.float32),
        pltpu.VMEM((NTP, 20), jnp.float32),
        pltpu.VMEM((NRP, 30), jnp.float32),
        pltpu.VMEM((NTP,), jnp.int32),
        pltpu.VMEM((32,), jnp.float32),
        pltpu.VMEM((NEP,), jnp.int32),
        pltpu.VMEM((5, C), jnp.int32),
        pltpu.VMEM((5, C), jnp.int32),
        pltpu.VMEM((C,), jnp.int32),
        pltpu.VMEM((C,), jnp.int32),
        pltpu.VMEM((C,), jnp.int32),
        pltpu.VMEM((C,), jnp.int32),
        pltpu.VMEM((C,), jnp.int32),
        pltpu.VMEM((C,), jnp.int32),
        pltpu.VMEM((C,), jnp.float32),
        pltpu.VMEM((C,), jnp.float32),
        pltpu.VMEM((C, 32), jnp.float32),
        pltpu.VMEM((C, 32), jnp.float32),
        pltpu.VMEM((C, 32), jnp.float32),
        pltpu.VMEM((C, 32), jnp.float32),
        pltpu.VMEM((C, ACC_W), jnp.float32),
        pltpu.VMEM((C, ACC_W), jnp.float32),
        pltpu.VMEM((RPT, ACC_W), jnp.float32),
        pltpu.SMEM((4,), jnp.int32),
        pltpu.SemaphoreType.DMA,
        pltpu.SemaphoreType.DMA,
        pltpu.SemaphoreType.DMA,
        pltpu.SemaphoreType.DMA,
        pltpu.SemaphoreType.DMA,
        pltpu.SemaphoreType.DMA,
    ],
)()


def kernel(head, relation, time, example_idx, dataset, rela_embed, time_embed,
           W_att1, W_att2, W_past, W_now, W_future, W_cls, b_cls):
    f32 = jnp.float32
    i32 = jnp.int32

    e_head = dataset[:, 0].astype(i32)
    e_rel = dataset[:, 1].astype(i32)
    e_tail = dataset[:, 2].astype(i32)
    e_time = dataset[:, 3].astype(i32)
    validi = jnp.ones((N_FACTS,), i32).at[example_idx].set(0)

    pad = NFP - N_FACTS
    pk = jnp.stack([
        jnp.pad(e_head, (0, pad)),
        jnp.pad(e_rel, (0, pad)),
        jnp.pad(e_tail, (0, pad)),
        jnp.pad(e_time, (0, pad)),
        jnp.pad(validi, (0, pad)),
    ])
    pk = pk.reshape(5, NFP // C, C).transpose(1, 0, 2)

    Ws = jnp.stack([W_past, W_now, W_future])
    W1h, W1r, W1q = W_att1[:D], W_att1[D:2 * D], W_att1[2 * D:]

    rw = jnp.einsum('rd,sde->rse', rela_embed, Ws).reshape((N_REL + 1) * 3, D)
    rw = jnp.pad(rw, ((0, 3 * NRP - (N_REL + 1) * 3), (0, 0)))

    dtt = jnp.arange(N_TIME)[None, :] - time[:, None]
    sidx = (jnp.sign(dtt) + 1).astype(i32)
    te_abs = time_embed[jnp.abs(dtt)]
    tb = jnp.einsum('btd,btde->bte', te_abs, Ws[sidx])
    tb = jnp.pad(tb, ((0, 0), (0, NTP - N_TIME), (0, 0)))
    st = jnp.pad(sidx, ((0, 0), (0, NTP - N_TIME)))

    ar = rela_embed @ W1r + (rela_embed[relation] @ W1q)[:, None, :]
    ar = jnp.pad(ar, ((0, 0), (0, NRP - (N_REL + 1)), (0, 0)))

    w2 = jnp.pad(W_att2[:, 0], (0, 2))

    w_cat = jnp.zeros((ACC_W, 128), f32)
    w_cat = w_cat.at[:D, 0:20].set(W_past)
    w_cat = w_cat.at[:D, 32:52].set(W_now)
    w_cat = w_cat.at[:D, 64:84].set(W_future)
    w_cat = w_cat.at[:D, 96:126].set(W1h)

    w_cls_p = jnp.zeros((ACC_W, 1), f32).at[:D].set(W_cls)

    acc = jnp.zeros((B, NEP, ACC_W), f32).at[jnp.arange(B), head, 20].set(1.0)

    for _ in range(3):
        t1, t2 = _build_tables(acc, w_cat)
        memv = (acc[:, :, 20] > 0).astype(i32)
        acc = _sc_edges(pk, t1, t2, rw, tb, ar, st, w2, memv)

    return _classifier(acc, w_cls_p, b_cls)


# speedup vs baseline: 43.2657x; 1.1909x over previous
import functools

import jax
import jax.numpy as jnp
from jax import lax
from jax.experimental import pallas as pl
from jax.experimental.pallas import tpu as pltpu
from jax.experimental.pallas import tpu_sc as plsc

N_ENT = 10000
N_REL = 230
N_TIME = 365
N_FACTS = 160000
B = 64
D = 20

NEP = 10240
C = 128
NFP = 163840
NTILE = 16
CPT = NFP // C // NTILE
HB = CPT // 2
BPC = B // 2
RPT = NEP // NTILE
NTP = 368
NRP = 232
GRP = C // 16
ACC_W = 24


def _build_body(acc_ref, w_ref, o_ref):
    a = acc_ref[0]
    h32 = jnp.where(a > 0, a, 0.01 * a)
    o = jnp.dot(h32, w_ref[...], preferred_element_type=jnp.float32)
    memf = (a[:, 20:21] > 0).astype(jnp.float32)
    iot = lax.broadcasted_iota(jnp.int32, (NEP, 128), 1)
    o_ref[0] = o + jnp.where(iot == 126, memf, 0.0)


def _build_tables(acc, w_cat):
    tab = pl.pallas_call(
        _build_body,
        grid=(B,),
        in_specs=[
            pl.BlockSpec((1, NEP, ACC_W), lambda b: (b, 0, 0)),
            pl.BlockSpec((ACC_W, 128), lambda b: (0, 0)),
        ],
        out_specs=pl.BlockSpec((1, NEP, 128), lambda b: (b, 0, 0)),
        out_shape=jax.ShapeDtypeStruct((B, NEP, 128), jnp.float32),
    )(acc, w_cat)
    t1 = tab[:, :, :96].reshape(B * NEP * 3, 32)
    t2 = tab[:, :, 96:].reshape(B * NEP, 32)
    return t1, t2


def _cls_body(acc_ref, w_ref, b_ref, o_ref):
    a = acc_ref[0]
    h32 = jnp.where(a > 0, a, 0.01 * a)
    r = jnp.dot(h32, w_ref[...], preferred_element_type=jnp.float32)
    memf = (a[:, 20:21] > 0).astype(jnp.float32)
    o_ref[0] = (r + b_ref[0, 0]) * memf


def _classifier(acc, w_cls_p, b_cls):
    out = pl.pallas_call(
        _cls_body,
        grid=(B,),
        in_specs=[
            pl.BlockSpec((1, NEP, ACC_W), lambda b: (b, 0, 0)),
            pl.BlockSpec((ACC_W, 1), lambda b: (0, 0)),
            pl.BlockSpec(memory_space=pltpu.SMEM),
        ],
        out_specs=pl.BlockSpec((1, NEP, 1), lambda b: (b, 0, 0)),
        out_shape=jax.ShapeDtypeStruct((B, NEP, 1), jnp.float32),
    )(acc, w_cls_p, b_cls.reshape(1, 1))
    return out[:, :N_ENT, 0]


def _sc_body(pk_hbm, t1_hbm, t2_hbm, rw_hbm, tb_hbm, ar_hbm, st_hbm, w2_hbm,
             memv_hbm, out_hbm,
             acc_sh, rw_v, tb_v, ar_v, st_v, w2_v, memv_v,
             pkb, i10, i20, i11, i21, el0, el1, mk0, mk1,
             t1r0, t2r0, t1r1, t2r1, msg0, msg1, zer_v, sflg,
             sg10, sg20, sg11, sg21, ssc0, ssc1):
    cid = lax.axis_index("c")
    sid = lax.axis_index("s")
    zeros16 = jnp.zeros((16,), jnp.float32)

    pltpu.sync_copy(rw_hbm, rw_v)
    pltpu.sync_copy(w2_hbm, w2_v)

    def _zrow(i, c):
        zer_v[i, pl.ds(0, 16)] = zeros16
        zer_v[i, pl.ds(8, 16)] = zeros16
        return c
    lax.fori_loop(0, RPT, _zrow, 0)

    def _mrow(i, c):
        msg0[i, pl.ds(8, 16)] = zeros16
        msg1[i, pl.ds(8, 16)] = zeros16
        return c
    lax.fori_loop(0, C, _mrow, 0)

    sflg[2] = 0
    sflg[3] = 0

    my_rows = pl.ds(sid * RPT, RPT)
    pltpu.sync_copy(zer_v, acc_sh.at[my_rows])
    plsc.subcore_barrier()

    iota16 = lax.iota(jnp.int32, 16)

    def _batch(bl, carry):
        b = cid * BPC + bl
        pltpu.sync_copy(tb_hbm.at[b], tb_v)
        pltpu.sync_copy(ar_hbm.at[b], ar_v)
        pltpu.sync_copy(st_hbm.at[b], st_v)
        pltpu.sync_copy(memv_hbm.at[b], memv_v)
        base1 = b * (NEP * 3)
        base2 = b * NEP

        def _fetch(rl, i1, i2, mk, p, t1r, t2r, s1, s2):
            def _idx(g, ma):
                o = g * 16
                sl = pl.ds(o, 16)
                ehd = pkb[rl, 0, sl]
                s16 = plsc.load_gather(st_v, [pkb[rl, 3, sl]])
                i1[sl] = ehd * 3 + s16 + base1
                i2[sl] = ehd + base2
                mf = (plsc.load_gather(memv_v, [ehd]) * pkb[rl, 4, sl]).astype(jnp.float32)
                mk[sl] = mf
                return jnp.maximum(ma, mf)
            ma = lax.fori_loop(0, GRP, _idx, zeros16)
            any_i = (jnp.max(ma) > 0).astype(jnp.int32)
            sflg[p] = any_i

            @pl.when(any_i > 0)
            def _():
                pltpu.async_copy(t1_hbm.at[i1], t1r, s1)
                pltpu.async_copy(t2_hbm.at[i2], t2r, s2)

        def _wait_g(i1, i2, t1r, t2r, s1, s2):
            pltpu.make_async_copy(t1_hbm.at[i1], t1r, s1).wait()
            pltpu.make_async_copy(t2_hbm.at[i2], t2r, s2).wait()

        def _compute(rl, mk, t1r, t2r, msg):
            def _grp(g, c3):
                o = g * 16
                sl = pl.ds(o, 16)
                mk16 = mk[sl]
                anyg = jnp.max(mk16) > 0

                @pl.when(anyg)
                def _():
                    row16 = iota16 + o
                    er16 = pkb[rl, 1, sl]
                    etm16 = pkb[rl, 3, sl]
                    s16 = plsc.load_gather(st_v, [etm16])
                    rr16 = er16 * 3 + s16
                    w2a = w2_v[pl.ds(0, 16)]
                    w2b = w2_v[pl.ds(16, 16)]
                    accs = jnp.zeros((16,), jnp.float32)
                    for k in range(30):
                        ck = jnp.full((16,), k, jnp.int32)
                        a = (plsc.load_gather(t2r, [row16, ck])
                             + plsc.load_gather(ar_v, [er16, ck]))
                        wk = w2a[k] if k < 16 else w2b[k - 16]
                        accs = accs + jnp.maximum(a, 0.0) * wk
                    sig = 1.0 / (1.0 + jnp.exp(-accs))
                    sm = sig * mk16
                    for k in range(20):
                        ck = jnp.full((16,), k, jnp.int32)
                        tr = (plsc.load_gather(t1r, [row16, ck])
                              + plsc.load_gather(rw_v, [rr16, ck])
                              + plsc.load_gather(tb_v, [etm16, ck]))
                        plsc.store_scatter(msg, [row16, ck], tr * sm)
                    plsc.store_scatter(msg, [row16, jnp.full((16,), 20, jnp.int32)], mk16)

                @pl.when(jnp.logical_not(anyg))
                def _():
                    for r in range(16):
                        msg[o + r, pl.ds(0, 16)] = zeros16
                        msg[o + r, pl.ds(8, 16)] = zeros16
                return c3
            lax.fori_loop(0, GRP, _grp, 0)

        def _fill_el(rl, el):
            def _f(g, c3):
                sl = pl.ds(g * 16, 16)
                el[sl] = pkb[rl, 2, sl]
                return c3
            lax.fori_loop(0, GRP, _f, 0)

        def _wait_sc(el, msg, s):
            pltpu.make_async_copy(msg, acc_sh.at[el], s).wait()

        for h in range(2):
            pltpu.sync_copy(pk_hbm.at[sid, pl.ds(h * HB, HB)], pkb)
            _fetch(0, i10, i20, mk0, 0, t1r0, t2r0, sg10, sg20)

            def _pair(j, c2):
                rl0 = 2 * j
                _fetch(rl0 + 1, i11, i21, mk1, 1, t1r1, t2r1, sg11, sg21)
                ga = sflg[0]

                @pl.when(sflg[2] > 0)
                def _():
                    _wait_sc(el0, msg0, ssc0)

                @pl.when(ga > 0)
                def _():
                    _wait_g(i10, i20, t1r0, t2r0, sg10, sg20)
                    _compute(rl0, mk0, t1r0, t2r0, msg0)
                    _fill_el(rl0, el0)
                    pltpu.async_copy(msg0, acc_sh.at[el0], ssc0, add=True)
                sflg[2] = ga

                @pl.when(j < HB // 2 - 1)
                def _():
                    _fetch(rl0 + 2, i10, i20, mk0, 0, t1r0, t2r0, sg10, sg20)
                gb = sflg[1]

                @pl.when(sflg[3] > 0)
                def _():
                    _wait_sc(el1, msg1, ssc1)

                @pl.when(gb > 0)
                def _():
                    _wait_g(i11, i21, t1r1, t2r1, sg11, sg21)
                    _compute(rl0 + 1, mk1, t1r1, t2r1, msg1)
                    _fill_el(rl0 + 1, el1)
                    pltpu.async_copy(msg1, acc_sh.at[el1], ssc1, add=True)
                sflg[3] = gb
                return c2
            lax.fori_loop(0, HB // 2, _pair, 0)
            @pl.when(sflg[2] > 0)
            def _():
                _wait_sc(el0, msg0, ssc0)

            @pl.when(sflg[3] > 0)
            def _():
                _wait_sc(el1, msg1, ssc1)
            sflg[2] = 0
            sflg[3] = 0

        plsc.subcore_barrier()
        pltpu.sync_copy(acc_sh.at[my_rows], out_hbm.at[b, my_rows])
        pltpu.sync_copy(zer_v, acc_sh.at[my_rows])
        plsc.subcore_barrier()
        return carry
    lax.fori_loop(0, BPC, _batch, 0)


_sc_edges = functools.partial(
    pl.kernel,
    _sc_body,
    out_type=jax.ShapeDtypeStruct((B, NEP, ACC_W), jnp.float32),
    mesh=plsc.VectorSubcoreMesh(core_axis_name="c", subcore_axis_name="s"),
    compiler_params=pltpu.CompilerParams(
        needs_layout_passes=False, use_tc_tiling_on_sc=False),
    scratch_types=[
        pltpu.VMEM_SHARED((NEP, ACC_W), jnp.float32),
        pltpu.VMEM((3 * NRP, 20), jnp.float32),
        pltpu.VMEM((NTP, 20), jnp.float32),
        pltpu.VMEM((NRP, 30), jnp.float32),
        pltpu.VMEM((NTP,), jnp.int32),
        pltpu.VMEM((32,), jnp.float32),
        pltpu.VMEM((NEP,), jnp.int32),
        pltpu.VMEM((HB, 5, C), jnp.int32),
        pltpu.VMEM((C,), jnp.int32),
        pltpu.VMEM((C,), jnp.int32),
        pltpu.VMEM((C,), jnp.int32),
        pltpu.VMEM((C,), jnp.int32),
        pltpu.VMEM((C,), jnp.int32),
        pltpu.VMEM((C,), jnp.int32),
        pltpu.VMEM((C,), jnp.float32),
        pltpu.VMEM((C,), jnp.float32),
        pltpu.VMEM((C, 32), jnp.float32),
        pltpu.VMEM((C, 32), jnp.float32),
        pltpu.VMEM((C, 32), jnp.float32),
        pltpu.VMEM((C, 32), jnp.float32),
        pltpu.VMEM((C, ACC_W), jnp.float32),
        pltpu.VMEM((C, ACC_W), jnp.float32),
        pltpu.VMEM((RPT, ACC_W), jnp.float32),
        pltpu.SMEM((4,), jnp.int32),
        pltpu.SemaphoreType.DMA,
        pltpu.SemaphoreType.DMA,
        pltpu.SemaphoreType.DMA,
        pltpu.SemaphoreType.DMA,
        pltpu.SemaphoreType.DMA,
        pltpu.SemaphoreType.DMA,
    ],
)()


def kernel(head, relation, time, example_idx, dataset, rela_embed, time_embed,
           W_att1, W_att2, W_past, W_now, W_future, W_cls, b_cls):
    f32 = jnp.float32
    i32 = jnp.int32

    e_head = dataset[:, 0].astype(i32)
    e_rel = dataset[:, 1].astype(i32)
    e_tail = dataset[:, 2].astype(i32)
    e_time = dataset[:, 3].astype(i32)
    validi = jnp.ones((N_FACTS,), i32).at[example_idx].set(0)

    pad = NFP - N_FACTS
    pk = jnp.stack([
        jnp.pad(e_head, (0, pad)),
        jnp.pad(e_rel, (0, pad)),
        jnp.pad(e_tail, (0, pad)),
        jnp.pad(e_time, (0, pad)),
        jnp.pad(validi, (0, pad)),
    ])
    pk = pk.reshape(5, NFP // C, C).transpose(1, 0, 2)
    pk = pk.reshape(CPT, NTILE, 5, C).transpose(1, 0, 2, 3)

    Ws = jnp.stack([W_past, W_now, W_future])
    W1h, W1r, W1q = W_att1[:D], W_att1[D:2 * D], W_att1[2 * D:]

    rw = jnp.einsum('rd,sde->rse', rela_embed, Ws).reshape((N_REL + 1) * 3, D)
    rw = jnp.pad(rw, ((0, 3 * NRP - (N_REL + 1) * 3), (0, 0)))

    dtt = jnp.arange(N_TIME)[None, :] - time[:, None]
    sidx = (jnp.sign(dtt) + 1).astype(i32)
    te_abs = time_embed[jnp.abs(dtt)]
    tb = jnp.einsum('btd,btde->bte', te_abs, Ws[sidx])
    tb = jnp.pad(tb, ((0, 0), (0, NTP - N_TIME), (0, 0)))
    st = jnp.pad(sidx, ((0, 0), (0, NTP - N_TIME)))

    ar = rela_embed @ W1r + (rela_embed[relation] @ W1q)[:, None, :]
    ar = jnp.pad(ar, ((0, 0), (0, NRP - (N_REL + 1)), (0, 0)))

    w2 = jnp.pad(W_att2[:, 0], (0, 2))

    w_cat = jnp.zeros((ACC_W, 128), f32)
    w_cat = w_cat.at[:D, 0:20].set(W_past)
    w_cat = w_cat.at[:D, 32:52].set(W_now)
    w_cat = w_cat.at[:D, 64:84].set(W_future)
    w_cat = w_cat.at[:D, 96:126].set(W1h)

    w_cls_p = jnp.zeros((ACC_W, 1), f32).at[:D].set(W_cls)

    acc = jnp.zeros((B, NEP, ACC_W), f32).at[jnp.arange(B), head, 20].set(1.0)

    for _ in range(3):
        t1, t2 = _build_tables(acc, w_cat)
        memv = (acc[:, :, 20] > 0).astype(i32)
        acc = _sc_edges(pk, t1, t2, rw, tb, ar, st, w2, memv)

    return _classifier(acc, w_cls_p, b_cls)


# speedup vs baseline: 44.2458x; 1.0227x over previous
import functools

import jax
import jax.numpy as jnp
from jax import lax
from jax.experimental import pallas as pl
from jax.experimental.pallas import tpu as pltpu
from jax.experimental.pallas import tpu_sc as plsc

N_ENT = 10000
N_REL = 230
N_TIME = 365
N_FACTS = 160000
B = 64
D = 20

NEP = 10240
C = 128
NFP = 163840
NTILE = 16
CPT = NFP // C // NTILE
HB = CPT // 2
BPC = B // 2
RPT = NEP // NTILE
NTP = 368
NRP = 232
GRP = C // 16
ACC_W = 24


def _build_body(acc_ref, w1_ref, w2_ref, o1_ref, o2_ref):
    a = acc_ref[0]
    h32 = jnp.where(a > 0, a, 0.01 * a)
    o1_ref[0] = jnp.dot(h32, w1_ref[...], preferred_element_type=jnp.float32)
    o2 = jnp.dot(h32, w2_ref[...], preferred_element_type=jnp.float32)
    memf = (a[:, 20:21] > 0).astype(jnp.float32)
    iot = lax.broadcasted_iota(jnp.int32, (NEP, 32), 1)
    o2_ref[0] = o2 + jnp.where(iot == 30, memf, 0.0)


def _build_tables(acc, w_cat):
    t1, t2 = pl.pallas_call(
        _build_body,
        grid=(B,),
        in_specs=[
            pl.BlockSpec((1, NEP, ACC_W), lambda b: (b, 0, 0)),
            pl.BlockSpec((ACC_W, 96), lambda b: (0, 0)),
            pl.BlockSpec((ACC_W, 32), lambda b: (0, 0)),
        ],
        out_specs=[
            pl.BlockSpec((1, NEP, 96), lambda b: (b, 0, 0)),
            pl.BlockSpec((1, NEP, 32), lambda b: (b, 0, 0)),
        ],
        out_shape=[
            jax.ShapeDtypeStruct((B, NEP, 96), jnp.float32),
            jax.ShapeDtypeStruct((B, NEP, 32), jnp.float32),
        ],
    )(acc, w_cat[:, :96], w_cat[:, 96:])
    return t1.reshape(B * NEP * 3, 32), t2.reshape(B * NEP, 32)


def _cls_body(acc_ref, w_ref, b_ref, o_ref):
    a = acc_ref[0]
    h32 = jnp.where(a > 0, a, 0.01 * a)
    r = jnp.dot(h32, w_ref[...], preferred_element_type=jnp.float32)
    memf = (a[:, 20:21] > 0).astype(jnp.float32)
    o_ref[0] = (r + b_ref[0, 0]) * memf


def _classifier(acc, w_cls_p, b_cls):
    out = pl.pallas_call(
        _cls_body,
        grid=(B,),
        in_specs=[
            pl.BlockSpec((1, NEP, ACC_W), lambda b: (b, 0, 0)),
            pl.BlockSpec((ACC_W, 1), lambda b: (0, 0)),
            pl.BlockSpec(memory_space=pltpu.SMEM),
        ],
        out_specs=pl.BlockSpec((1, NEP, 1), lambda b: (b, 0, 0)),
        out_shape=jax.ShapeDtypeStruct((B, NEP, 1), jnp.float32),
    )(acc, w_cls_p, b_cls.reshape(1, 1))
    return out[:, :N_ENT, 0]


def _sc_body(pk_hbm, t1_hbm, t2_hbm, rw_hbm, tb_hbm, ar_hbm, st_hbm, w2_hbm,
             memv_hbm, out_hbm,
             acc_sh, rw_v, tb_v, ar_v, st_v, w2_v, memv_v,
             pkb, i10, i20, i11, i21, el0, el1, mk0, mk1,
             t1r0, t2r0, t1r1, t2r1, msg0, msg1, zer_v, sflg,
             sg10, sg20, sg11, sg21, ssc0, ssc1):
    cid = lax.axis_index("c")
    sid = lax.axis_index("s")
    zeros16 = jnp.zeros((16,), jnp.float32)

    pltpu.sync_copy(rw_hbm, rw_v)
    pltpu.sync_copy(w2_hbm, w2_v)

    def _zrow(i, c):
        zer_v[i, pl.ds(0, 16)] = zeros16
        zer_v[i, pl.ds(8, 16)] = zeros16
        return c
    lax.fori_loop(0, RPT, _zrow, 0)

    def _mrow(i, c):
        msg0[i, pl.ds(8, 16)] = zeros16
        msg1[i, pl.ds(8, 16)] = zeros16
        return c
    lax.fori_loop(0, C, _mrow, 0)

    sflg[2] = 0
    sflg[3] = 0

    my_rows = pl.ds(sid * RPT, RPT)
    pltpu.sync_copy(zer_v, acc_sh.at[my_rows])
    plsc.subcore_barrier()

    iota16 = lax.iota(jnp.int32, 16)

    def _batch(bl, carry):
        b = cid * BPC + bl
        pltpu.sync_copy(tb_hbm.at[b], tb_v)
        pltpu.sync_copy(ar_hbm.at[b], ar_v)
        pltpu.sync_copy(st_hbm.at[b], st_v)
        pltpu.sync_copy(memv_hbm.at[b], memv_v)
        base1 = b * (NEP * 3)
        base2 = b * NEP

        def _fetch(rl, i1, i2, mk, p, t1r, t2r, s1, s2):
            def _idx(g, ma):
                o = g * 16
                sl = pl.ds(o, 16)
                ehd = pkb[rl, 0, sl]
                s16 = plsc.load_gather(st_v, [pkb[rl, 3, sl]])
                i1[sl] = ehd * 3 + s16 + base1
                i2[sl] = ehd + base2
                mf = (plsc.load_gather(memv_v, [ehd]) * pkb[rl, 4, sl]).astype(jnp.float32)
                mk[sl] = mf
                return jnp.maximum(ma, mf)
            ma = lax.fori_loop(0, GRP, _idx, zeros16)
            any_i = (jnp.max(ma) > 0).astype(jnp.int32)
            sflg[p] = any_i

            @pl.when(any_i > 0)
            def _():
                pltpu.async_copy(t1_hbm.at[i1], t1r, s1)
                pltpu.async_copy(t2_hbm.at[i2], t2r, s2)

        def _wait_g(i1, i2, t1r, t2r, s1, s2):
            pltpu.make_async_copy(t1_hbm.at[i1], t1r, s1).wait()
            pltpu.make_async_copy(t2_hbm.at[i2], t2r, s2).wait()

        def _compute(rl, mk, t1r, t2r, msg):
            def _grp(g, c3):
                o = g * 16
                sl = pl.ds(o, 16)
                mk16 = mk[sl]
                anyg = jnp.max(mk16) > 0

                @pl.when(anyg)
                def _():
                    row16 = iota16 + o
                    er16 = pkb[rl, 1, sl]
                    etm16 = pkb[rl, 3, sl]
                    s16 = plsc.load_gather(st_v, [etm16])
                    rr16 = er16 * 3 + s16
                    w2a = w2_v[pl.ds(0, 16)]
                    w2b = w2_v[pl.ds(16, 16)]
                    accs = jnp.zeros((16,), jnp.float32)
                    for k in range(30):
                        ck = jnp.full((16,), k, jnp.int32)
                        a = (plsc.load_gather(t2r, [row16, ck])
                             + plsc.load_gather(ar_v, [er16, ck]))
                        wk = w2a[k] if k < 16 else w2b[k - 16]
                        accs = accs + jnp.maximum(a, 0.0) * wk
                    sig = 1.0 / (1.0 + jnp.exp(-accs))
                    sm = sig * mk16
                    for k in range(20):
                        ck = jnp.full((16,), k, jnp.int32)
                        tr = (plsc.load_gather(t1r, [row16, ck])
                              + plsc.load_gather(rw_v, [rr16, ck])
                              + plsc.load_gather(tb_v, [etm16, ck]))
                        plsc.store_scatter(msg, [row16, ck], tr * sm)
                    plsc.store_scatter(msg, [row16, jnp.full((16,), 20, jnp.int32)], mk16)

                @pl.when(jnp.logical_not(anyg))
                def _():
                    for r in range(16):
                        msg[o + r, pl.ds(0, 16)] = zeros16
                        msg[o + r, pl.ds(8, 16)] = zeros16
                return c3
            lax.fori_loop(0, GRP, _grp, 0)

        def _fill_el(rl, el):
            def _f(g, c3):
                sl = pl.ds(g * 16, 16)
                el[sl] = pkb[rl, 2, sl]
                return c3
            lax.fori_loop(0, GRP, _f, 0)

        def _wait_sc(el, msg, s):
            pltpu.make_async_copy(msg, acc_sh.at[el], s).wait()

        for h in range(2):
            pltpu.sync_copy(pk_hbm.at[sid, pl.ds(h * HB, HB)], pkb)
            _fetch(0, i10, i20, mk0, 0, t1r0, t2r0, sg10, sg20)

            def _pair(j, c2):
                rl0 = 2 * j
                _fetch(rl0 + 1, i11, i21, mk1, 1, t1r1, t2r1, sg11, sg21)
                ga = sflg[0]

                @pl.when(sflg[2] > 0)
                def _():
                    _wait_sc(el0, msg0, ssc0)

                @pl.when(ga > 0)
                def _():
                    _wait_g(i10, i20, t1r0, t2r0, sg10, sg20)
                    _compute(rl0, mk0, t1r0, t2r0, msg0)
                    _fill_el(rl0, el0)
                    pltpu.async_copy(msg0, acc_sh.at[el0], ssc0, add=True)
                sflg[2] = ga

                @pl.when(j < HB // 2 - 1)
                def _():
                    _fetch(rl0 + 2, i10, i20, mk0, 0, t1r0, t2r0, sg10, sg20)
                gb = sflg[1]

                @pl.when(sflg[3] > 0)
                def _():
                    _wait_sc(el1, msg1, ssc1)

                @pl.when(gb > 0)
                def _():
                    _wait_g(i11, i21, t1r1, t2r1, sg11, sg21)
                    _compute(rl0 + 1, mk1, t1r1, t2r1, msg1)
                    _fill_el(rl0 + 1, el1)
                    pltpu.async_copy(msg1, acc_sh.at[el1], ssc1, add=True)
                sflg[3] = gb
                return c2
            lax.fori_loop(0, HB // 2, _pair, 0)
            @pl.when(sflg[2] > 0)
            def _():
                _wait_sc(el0, msg0, ssc0)

            @pl.when(sflg[3] > 0)
            def _():
                _wait_sc(el1, msg1, ssc1)
            sflg[2] = 0
            sflg[3] = 0

        plsc.subcore_barrier()
        pltpu.sync_copy(acc_sh.at[my_rows], out_hbm.at[b, my_rows])
        pltpu.sync_copy(zer_v, acc_sh.at[my_rows])
        plsc.subcore_barrier()
        return carry
    lax.fori_loop(0, BPC, _batch, 0)


_sc_edges = functools.partial(
    pl.kernel,
    _sc_body,
    out_type=jax.ShapeDtypeStruct((B, NEP, ACC_W), jnp.float32),
    mesh=plsc.VectorSubcoreMesh(core_axis_name="c", subcore_axis_name="s"),
    compiler_params=pltpu.CompilerParams(
        needs_layout_passes=False, use_tc_tiling_on_sc=False),
    scratch_types=[
        pltpu.VMEM_SHARED((NEP, ACC_W), jnp.float32),
        pltpu.VMEM((3 * NRP, 20), jnp.float32),
        pltpu.VMEM((NTP, 20), jnp.float32),
        pltpu.VMEM((NRP, 30), jnp.float32),
        pltpu.VMEM((NTP,), jnp.int32),
        pltpu.VMEM((32,), jnp.float32),
        pltpu.VMEM((NEP,), jnp.int32),
        pltpu.VMEM((HB, 5, C), jnp.int32),
        pltpu.VMEM((C,), jnp.int32),
        pltpu.VMEM((C,), jnp.int32),
        pltpu.VMEM((C,), jnp.int32),
        pltpu.VMEM((C,), jnp.int32),
        pltpu.VMEM((C,), jnp.int32),
        pltpu.VMEM((C,), jnp.int32),
        pltpu.VMEM((C,), jnp.float32),
        pltpu.VMEM((C,), jnp.float32),
        pltpu.VMEM((C, 32), jnp.float32),
        pltpu.VMEM((C, 32), jnp.float32),
        pltpu.VMEM((C, 32), jnp.float32),
        pltpu.VMEM((C, 32), jnp.float32),
        pltpu.VMEM((C, ACC_W), jnp.float32),
        pltpu.VMEM((C, ACC_W), jnp.float32),
        pltpu.VMEM((RPT, ACC_W), jnp.float32),
        pltpu.SMEM((4,), jnp.int32),
        pltpu.SemaphoreType.DMA,
        pltpu.SemaphoreType.DMA,
        pltpu.SemaphoreType.DMA,
        pltpu.SemaphoreType.DMA,
        pltpu.SemaphoreType.DMA,
        pltpu.SemaphoreType.DMA,
    ],
)()


def kernel(head, relation, time, example_idx, dataset, rela_embed, time_embed,
           W_att1, W_att2, W_past, W_now, W_future, W_cls, b_cls):
    f32 = jnp.float32
    i32 = jnp.int32

    e_head = dataset[:, 0].astype(i32)
    e_rel = dataset[:, 1].astype(i32)
    e_tail = dataset[:, 2].astype(i32)
    e_time = dataset[:, 3].astype(i32)
    validi = jnp.ones((N_FACTS,), i32).at[example_idx].set(0)

    pad = NFP - N_FACTS
    pk = jnp.stack([
        jnp.pad(e_head, (0, pad)),
        jnp.pad(e_rel, (0, pad)),
        jnp.pad(e_tail, (0, pad)),
        jnp.pad(e_time, (0, pad)),
        jnp.pad(validi, (0, pad)),
    ])
    pk = pk.reshape(5, NFP // C, C).transpose(1, 0, 2)
    pk = pk.reshape(CPT, NTILE, 5, C).transpose(1, 0, 2, 3)

    Ws = jnp.stack([W_past, W_now, W_future])
    W1h, W1r, W1q = W_att1[:D], W_att1[D:2 * D], W_att1[2 * D:]

    rw = jnp.einsum('rd,sde->rse', rela_embed, Ws).reshape((N_REL + 1) * 3, D)
    rw = jnp.pad(rw, ((0, 3 * NRP - (N_REL + 1) * 3), (0, 0)))

    dtt = jnp.arange(N_TIME)[None, :] - time[:, None]
    sidx = (jnp.sign(dtt) + 1).astype(i32)
    te_abs = time_embed[jnp.abs(dtt)]
    tb = jnp.einsum('btd,btde->bte', te_abs, Ws[sidx])
    tb = jnp.pad(tb, ((0, 0), (0, NTP - N_TIME), (0, 0)))
    st = jnp.pad(sidx, ((0, 0), (0, NTP - N_TIME)))

    ar = rela_embed @ W1r + (rela_embed[relation] @ W1q)[:, None, :]
    ar = jnp.pad(ar, ((0, 0), (0, NRP - (N_REL + 1)), (0, 0)))

    w2 = jnp.pad(W_att2[:, 0], (0, 2))

    w_cat = jnp.zeros((ACC_W, 128), f32)
    w_cat = w_cat.at[:D, 0:20].set(W_past)
    w_cat = w_cat.at[:D, 32:52].set(W_now)
    w_cat = w_cat.at[:D, 64:84].set(W_future)
    w_cat = w_cat.at[:D, 96:126].set(W1h)

    w_cls_p = jnp.zeros((ACC_W, 1), f32).at[:D].set(W_cls)

    acc = jnp.zeros((B, NEP, ACC_W), f32).at[jnp.arange(B), head, 20].set(1.0)

    for _ in range(3):
        t1, t2 = _build_tables(acc, w_cat)
        memv = (acc[:, :, 20] > 0).astype(i32)
        acc = _sc_edges(pk, t1, t2, rw, tb, ar, st, w2, memv)

    return _classifier(acc, w_cls_p, b_cls)
